# baseline (speedup 1.0000x reference)
import functools

import jax
import jax.numpy as jnp
from jax import lax
from jax.experimental import pallas as pl
from jax.experimental.pallas import tpu as pltpu
from jax.experimental.pallas import tpu_sc as plsc

P = 10000
E = 320000
D = 128
W = D // 2
NC, NS, L = 2, 16, 16
NW = NC * NS

P_PAD = 10240
ROWS_W = P_PAD // NW
RSUB = 80
EW = E // NW
EC = 80
NCHUNK = EW // EC


def _mesh():
    return plsc.VectorSubcoreMesh(core_axis_name="c", subcore_axis_name="s")


def _wid():
    return lax.axis_index("s") * NC + lax.axis_index("c")


@functools.partial(
    pl.kernel,
    mesh=_mesh(),
    compiler_params=pltpu.CompilerParams(needs_layout_passes=False, use_tc_tiling_on_sc=False),
    out_type=jax.ShapeDtypeStruct((P_PAD, W), jnp.int32),
    scratch_types=[
        pltpu.VMEM((ROWS_W,), jnp.int32),
        pltpu.VMEM((ROWS_W,), jnp.int32),
        pltpu.VMEM((RSUB, W), jnp.int32),
        pltpu.VMEM((RSUB, W), jnp.int32),
        pltpu.SemaphoreType.DMA,
        pltpu.SemaphoreType.DMA,
    ],
)
def _hplay_kernel(emb, s0, s1, hp, i0_v, i1_v, a_v, b_v, sem_a, sem_b):
    wid = _wid()
    base = wid * ROWS_W
    pltpu.sync_copy(s0.at[pl.ds(base, ROWS_W)], i0_v)
    pltpu.sync_copy(s1.at[pl.ds(base, ROWS_W)], i1_v)
    half = jnp.full((2 * L,), 0.5, jnp.bfloat16)
    for sub in range(ROWS_W // RSUB):
        ca = pltpu.async_copy(emb.at[i0_v.at[pl.ds(sub * RSUB, RSUB)]], a_v, sem_a)
        cb = pltpu.async_copy(emb.at[i1_v.at[pl.ds(sub * RSUB, RSUB)]], b_v, sem_b)
        ca.wait()
        cb.wait()

        def row_mean(r, _):
            for w4 in range(W // L):
                sl = pl.ds(w4 * L, L)
                a_bf = plsc.bitcast(a_v[r, sl], jnp.bfloat16)
                b_bf = plsc.bitcast(b_v[r, sl], jnp.bfloat16)
                a_v[r, sl] = plsc.bitcast((a_bf + b_bf) * half, jnp.int32)
            return 0

        lax.fori_loop(0, RSUB, row_mean, 0)
        pltpu.sync_copy(a_v, hp.at[pl.ds(base + sub * RSUB, RSUB)])


@functools.partial(
    pl.kernel,
    mesh=_mesh(),
    compiler_params=pltpu.CompilerParams(needs_layout_passes=False, use_tc_tiling_on_sc=False),
    out_type=jax.ShapeDtypeStruct((E,), jnp.float32),
    scratch_types=[
        pltpu.VMEM((EW,), jnp.int32),
        pltpu.VMEM((EW,), jnp.int32),
        pltpu.VMEM((EW,), jnp.float32),
        *([pltpu.VMEM((EC, W), jnp.int32)] * 8),
        *([pltpu.SemaphoreType.DMA] * 8),
    ],
)
def _score_kernel(hp, emb, src, dst, out, src_v, dst_v, sc_v,
                  a0, a1, a2, a3, b0, b1, b2, b3,
                  sa0, sa1, sa2, sa3, sb0, sb1, sb2, sb3):
    wid = _wid()
    eb = wid * EW
    pltpu.sync_copy(src.at[pl.ds(eb, EW)], src_v)
    pltpu.sync_copy(dst.at[pl.ds(eb, EW)], dst_v)

    a_bufs, b_bufs = (a0, a1, a2, a3), (b0, b1, b2, b3)
    a_sems, b_sems = (sa0, sa1, sa2, sa3), (sb0, sb1, sb2, sb3)
    NBUF = 4

    def idx_a(c):
        return src_v.at[pl.ds(pl.multiple_of(c * EC, 8), EC)]

    def idx_b(c):
        return dst_v.at[pl.ds(pl.multiple_of(c * EC, 8), EC)]

    def issue(c, u):
        pltpu.async_copy(hp.at[idx_a(c)], a_bufs[u], a_sems[u])
        pltpu.async_copy(emb.at[idx_b(c)], b_bufs[u], b_sems[u])

    def wait(c, u):
        pltpu.make_async_copy(hp.at[idx_a(c)], a_bufs[u], a_sems[u]).wait()
        pltpu.make_async_copy(emb.at[idx_b(c)], b_bufs[u], b_sems[u]).wait()

    def compute(c, a_v, b_v):
        off = pl.multiple_of(c * EC, 8)
        lane = lax.iota(jnp.int32, L)
        for g in range(EC // L):
            rows = lane + g * L
            acc0 = jnp.zeros((L,), jnp.float32)
            acc1 = jnp.zeros((L,), jnp.float32)

            def wstep(w8, accs):
                acc0, acc1 = accs
                for uu in range(8):
                    cols = (lane + (w8 * 8 + uu)) & (W - 1)
                    wa = plsc.load_gather(a_v, [rows, cols])
                    wb = plsc.load_gather(b_v, [rows, cols])
                    a_lo, a_hi = plsc.unpack(
                        plsc.bitcast(wa, jnp.bfloat16),
                        format=plsc.PackFormat.INTERLEAVED)
                    b_lo, b_hi = plsc.unpack(
                        plsc.bitcast(wb, jnp.bfloat16),
                        format=plsc.PackFormat.INTERLEAVED)
                    acc0 = acc0 + a_lo * b_lo
                    acc1 = acc1 + a_hi * b_hi
                return acc0, acc1

            acc0, acc1 = lax.fori_loop(0, W // 8, wstep, (acc0, acc1))
            sc_v[pl.ds(off + g * L, L)] = acc0 + acc1

    for j in range(NBUF):
        issue(j, j)

    def ring(i2, _):
        for u in range(NBUF):
            c = i2 * NBUF + u
            wait(c, u)
            compute(c, a_bufs[u], b_bufs[u])

            @pl.when(c + NBUF < NCHUNK)
            def _():
                issue(c + NBUF, u)
        return 0

    lax.fori_loop(0, (NCHUNK - 1) // NBUF, ring, 0)
    last = NCHUNK - 1
    wait(last, last % NBUF)
    compute(last, a_bufs[last % NBUF], b_bufs[last % NBUF])
    pltpu.sync_copy(sc_v, out.at[pl.ds(eb, EW)])


def kernel(track_emb, edge_index, sampled_tracks):
    t_bf = track_emb.astype(jnp.bfloat16)
    t_i32 = lax.bitcast_convert_type(t_bf.reshape(-1, W, 2), jnp.int32)
    src = edge_index[0].astype(jnp.int32)
    dst = edge_index[1].astype(jnp.int32)
    st = sampled_tracks.astype(jnp.int32)
    s0 = jnp.pad(st[:, 0], (0, P_PAD - P))
    s1 = jnp.pad(st[:, 1], (0, P_PAD - P))
    hp = _hplay_kernel(t_i32, s0, s1)
    return _score_kernel(hp, t_i32, src, dst)


# speedup vs baseline: 1.1730x; 1.1730x over previous
import functools

import jax
import jax.numpy as jnp
from jax import lax
from jax.experimental import pallas as pl
from jax.experimental.pallas import tpu as pltpu
from jax.experimental.pallas import tpu_sc as plsc

P = 10000
T = 10000
E = 320000
D = 128
W = D // 2
NC, NS, L = 2, 16, 16
NW = NC * NS

PK_CH = 25
PK_SUB = T // NS

P_PAD = 10240
HP_SUB = P_PAD // NS
RSUB = 80

EW = E // NW
EC = 80
NCHUNK = EW // EC
NBUF = 2


@functools.partial(
    pl.kernel,
    mesh=plsc.VectorSubcoreMesh(core_axis_name="c", subcore_axis_name="s"),
    compiler_params=pltpu.CompilerParams(needs_layout_passes=False,
                                         use_tc_tiling_on_sc=False),
    out_type=jax.ShapeDtypeStruct((E,), jnp.float32),
    scratch_types=[
        pltpu.VMEM_SHARED((T, W), jnp.int32),
        pltpu.VMEM_SHARED((P_PAD, W), jnp.int32),
        pltpu.VMEM((PK_CH, D), jnp.float32),
        pltpu.VMEM((PK_CH, W), jnp.int32),
        pltpu.VMEM((HP_SUB,), jnp.int32),
        pltpu.VMEM((HP_SUB,), jnp.int32),
        pltpu.VMEM((EW,), jnp.int32),
        pltpu.VMEM((EW,), jnp.int32),
        *([pltpu.VMEM((EC, W), jnp.int32)] * 4),
        *([pltpu.VMEM((EC,), jnp.float32)] * 2),
        *([pltpu.SemaphoreType.DMA] * 6),
    ],
)
def _fused_kernel(emb, s0, s1, src, dst, out,
                  emb_s, hp_s, pk_f, pk_w, i0_v, i1_v, src_v, dst_v,
                  a0, a1, b0, b1, so0, so1,
                  sa0, sa1, sb0, sb1, so_s0, so_s1):
    sid = lax.axis_index("s")
    cid = lax.axis_index("c")
    wid = sid * NC + cid

    for k in range(PK_SUB // PK_CH):
        r0 = sid * PK_SUB + k * PK_CH
        pltpu.sync_copy(emb.at[pl.ds(r0, PK_CH)], pk_f)

        def prow(r, _):
            for q in range(D // (2 * L)):
                pair = plsc.pack(pk_f[r, pl.ds(q * 2 * L, L)],
                                 pk_f[r, pl.ds(q * 2 * L + L, L)],
                                 format=plsc.PackFormat.INTERLEAVED)
                pk_w[r, pl.ds(q * L, L)] = plsc.bitcast(pair, jnp.int32)
            return 0

        lax.fori_loop(0, PK_CH, prow, 0)
        pltpu.sync_copy(pk_w, emb_s.at[pl.ds(r0, PK_CH)])
    plsc.subcore_barrier()

    hb = sid * HP_SUB
    pltpu.sync_copy(s0.at[pl.ds(hb, HP_SUB)], i0_v)
    pltpu.sync_copy(s1.at[pl.ds(hb, HP_SUB)], i1_v)
    half = jnp.full((2 * L,), 0.5, jnp.bfloat16)
    for k in range(HP_SUB // RSUB):
        ca = pltpu.async_copy(emb_s.at[i0_v.at[pl.ds(k * RSUB, RSUB)]], a0, sa0)
        cb = pltpu.async_copy(emb_s.at[i1_v.at[pl.ds(k * RSUB, RSUB)]], b0, sb0)
        ca.wait()
        cb.wait()

        def hrow(r, _):
            for q in range(W // L):
                sl = pl.ds(q * L, L)
                m = (plsc.bitcast(a0[r, sl], jnp.bfloat16) +
                     plsc.bitcast(b0[r, sl], jnp.bfloat16)) * half
                a0[r, sl] = plsc.bitcast(m, jnp.int32)
            return 0

        lax.fori_loop(0, RSUB, hrow, 0)
        pltpu.sync_copy(a0, hp_s.at[pl.ds(hb + k * RSUB, RSUB)])
    plsc.subcore_barrier()

    eb = wid * EW
    pltpu.sync_copy(src.at[pl.ds(eb, EW)], src_v)
    pltpu.sync_copy(dst.at[pl.ds(eb, EW)], dst_v)

    a_bufs, b_bufs = (a0, a1), (b0, b1)
    a_sems, b_sems = (sa0, sa1), (sb0, sb1)
    so_bufs, so_sems = (so0, so1), (so_s0, so_s1)

    def idx_a(c):
        return src_v.at[pl.ds(pl.multiple_of(c * EC, 8), EC)]

    def idx_b(c):
        return dst_v.at[pl.ds(pl.multiple_of(c * EC, 8), EC)]

    def out_at(c):
        return out.at[pl.ds(eb + pl.multiple_of(c * EC, 8), EC)]

    def issue(c, u):
        pltpu.async_copy(hp_s.at[idx_a(c)], a_bufs[u], a_sems[u])
        pltpu.async_copy(emb_s.at[idx_b(c)], b_bufs[u], b_sems[u])

    def wait(c, u):
        pltpu.make_async_copy(hp_s.at[idx_a(c)], a_bufs[u], a_sems[u]).wait()
        pltpu.make_async_copy(emb_s.at[idx_b(c)], b_bufs[u], b_sems[u]).wait()

    def compute(c, u):
        a_v, b_v = a_bufs[u], b_bufs[u]
        lane = lax.iota(jnp.int32, L)
        for g in range(EC // L):
            rows = lane + g * L
            acc0 = jnp.zeros((L,), jnp.float32)
            acc1 = jnp.zeros((L,), jnp.float32)

            def wstep(w8, accs):
                acc0, acc1 = accs
                for uu in range(8):
                    cols = (lane + (w8 * 8 + uu)) & (W - 1)
                    wa = plsc.load_gather(a_v, [rows, cols])
                    wb = plsc.load_gather(b_v, [rows, cols])
                    a_lo, a_hi = plsc.unpack(
                        plsc.bitcast(wa, jnp.bfloat16),
                        format=plsc.PackFormat.INTERLEAVED)
                    b_lo, b_hi = plsc.unpack(
                        plsc.bitcast(wb, jnp.bfloat16),
                        format=plsc.PackFormat.INTERLEAVED)
                    acc0 = acc0 + a_lo * b_lo
                    acc1 = acc1 + a_hi * b_hi
                return acc0, acc1

            acc0, acc1 = lax.fori_loop(0, W // 8, wstep, (acc0, acc1))
            so_bufs[u][pl.ds(g * L, L)] = acc0 + acc1

    def body(c, u, static):
        wait(c, u)
        if static:
            if c >= NBUF:
                pltpu.make_async_copy(so_bufs[u], out_at(c - NBUF),
                                      so_sems[u]).wait()
        else:
            @pl.when(c >= NBUF)
            def _():
                pltpu.make_async_copy(so_bufs[u], out_at(c - NBUF),
                                      so_sems[u]).wait()
        compute(c, u)
        pltpu.async_copy(so_bufs[u], out_at(c), so_sems[u])
        if static:
            if c + NBUF < NCHUNK:
                issue(c + NBUF, u)
        else:
            @pl.when(c + NBUF < NCHUNK)
            def _():
                issue(c + NBUF, u)

    for j in range(NBUF):
        issue(j, j)

    FI = NCHUNK // NBUF - 1

    def ring(i2, _):
        for u in range(NBUF):
            body(i2 * NBUF + u, u, static=False)
        return 0

    lax.fori_loop(0, FI, ring, 0)
    for c in range(FI * NBUF, NCHUNK):
        body(c, c % NBUF, static=True)
    for u in range(NBUF):
        c_last = ((NCHUNK - 1 - u) // NBUF) * NBUF + u
        pltpu.make_async_copy(so_bufs[u], out_at(c_last), so_sems[u]).wait()


def kernel(track_emb, edge_index, sampled_tracks):
    track_emb = track_emb.astype(jnp.float32)
    src = edge_index[0].astype(jnp.int32)
    dst = edge_index[1].astype(jnp.int32)
    st = sampled_tracks.astype(jnp.int32)
    s0 = jnp.pad(st[:, 0], (0, P_PAD - P))
    s1 = jnp.pad(st[:, 1], (0, P_PAD - P))
    return _fused_kernel(track_emb, s0, s1, src, dst)


# speedup vs baseline: 1.1733x; 1.0003x over previous
import functools

import jax
import jax.numpy as jnp
from jax import lax
from jax.experimental import pallas as pl
from jax.experimental.pallas import tpu as pltpu
from jax.experimental.pallas import tpu_sc as plsc

P = 10000
T = 10000
E = 320000
D = 128
W = D // 2
NC, NS, L = 2, 16, 16
NW = NC * NS

PK_CH = 25
PK_SUB = T // NS

P_PAD = 10240
HP_SUB = P_PAD // NS
RSUB = 80

EW = E // NW
EC = 80
NCHUNK = EW // EC
NBUF = 2


@functools.partial(
    pl.kernel,
    mesh=plsc.VectorSubcoreMesh(core_axis_name="c", subcore_axis_name="s"),
    compiler_params=pltpu.CompilerParams(needs_layout_passes=False,
                                         use_tc_tiling_on_sc=False),
    out_type=jax.ShapeDtypeStruct((E,), jnp.float32),
    scratch_types=[
        pltpu.VMEM_SHARED((T, W), jnp.int32),
        pltpu.VMEM_SHARED((P_PAD, W), jnp.int32),
        pltpu.VMEM((PK_CH, D), jnp.float32),
        pltpu.VMEM((PK_CH, W), jnp.int32),
        pltpu.VMEM((HP_SUB,), jnp.int32),
        pltpu.VMEM((HP_SUB,), jnp.int32),
        pltpu.VMEM((EW,), jnp.int32),
        pltpu.VMEM((EW,), jnp.int32),
        *([pltpu.VMEM((EC, W), jnp.int32)] * 4),
        *([pltpu.VMEM((EC,), jnp.float32)] * 2),
        *([pltpu.SemaphoreType.DMA] * 6),
    ],
)
def _fused_kernel(emb, s0, s1, src, dst, out,
                  emb_s, hp_s, pk_f, pk_w, i0_v, i1_v, src_v, dst_v,
                  a0, a1, b0, b1, so0, so1,
                  sa0, sa1, sb0, sb1, so_s0, so_s1):
    sid = lax.axis_index("s")
    cid = lax.axis_index("c")
    wid = sid * NC + cid

    _sc1 = jax.named_scope("pack_phase"); _sc1.__enter__()
    for k in range(PK_SUB // PK_CH):
        r0 = sid * PK_SUB + k * PK_CH
        pltpu.sync_copy(emb.at[pl.ds(r0, PK_CH)], pk_f)

        def prow(r, _):
            for q in range(D // (2 * L)):
                pair = plsc.pack(pk_f[r, pl.ds(q * 2 * L, L)],
                                 pk_f[r, pl.ds(q * 2 * L + L, L)],
                                 format=plsc.PackFormat.INTERLEAVED)
                pk_w[r, pl.ds(q * L, L)] = plsc.bitcast(pair, jnp.int32)
            return 0

        lax.fori_loop(0, PK_CH, prow, 0)
        pltpu.sync_copy(pk_w, emb_s.at[pl.ds(r0, PK_CH)])
    _sc1.__exit__(None, None, None)
    plsc.subcore_barrier()
    _sc2 = jax.named_scope("hplay_phase"); _sc2.__enter__()

    hb = sid * HP_SUB
    pltpu.sync_copy(s0.at[pl.ds(hb, HP_SUB)], i0_v)
    pltpu.sync_copy(s1.at[pl.ds(hb, HP_SUB)], i1_v)
    half = jnp.full((2 * L,), 0.5, jnp.bfloat16)
    for k in range(HP_SUB // RSUB):
        ca = pltpu.async_copy(emb_s.at[i0_v.at[pl.ds(k * RSUB, RSUB)]], a0, sa0)
        cb = pltpu.async_copy(emb_s.at[i1_v.at[pl.ds(k * RSUB, RSUB)]], b0, sb0)
        ca.wait()
        cb.wait()

        def hrow(r, _):
            for q in range(W // L):
                sl = pl.ds(q * L, L)
                m = (plsc.bitcast(a0[r, sl], jnp.bfloat16) +
                     plsc.bitcast(b0[r, sl], jnp.bfloat16)) * half
                a0[r, sl] = plsc.bitcast(m, jnp.int32)
            return 0

        lax.fori_loop(0, RSUB, hrow, 0)
        pltpu.sync_copy(a0, hp_s.at[pl.ds(hb + k * RSUB, RSUB)])
    _sc2.__exit__(None, None, None)
    plsc.subcore_barrier()
    _sc3 = jax.named_scope("score_phase"); _sc3.__enter__()

    eb = wid * EW
    pltpu.sync_copy(src.at[pl.ds(eb, EW)], src_v)
    pltpu.sync_copy(dst.at[pl.ds(eb, EW)], dst_v)

    a_bufs, b_bufs = (a0, a1), (b0, b1)
    a_sems, b_sems = (sa0, sa1), (sb0, sb1)
    so_bufs, so_sems = (so0, so1), (so_s0, so_s1)

    def idx_a(c):
        return src_v.at[pl.ds(pl.multiple_of(c * EC, 8), EC)]

    def idx_b(c):
        return dst_v.at[pl.ds(pl.multiple_of(c * EC, 8), EC)]

    def out_at(c):
        return out.at[pl.ds(eb + pl.multiple_of(c * EC, 8), EC)]

    def issue(c, u):
        pltpu.async_copy(hp_s.at[idx_a(c)], a_bufs[u], a_sems[u])
        pltpu.async_copy(emb_s.at[idx_b(c)], b_bufs[u], b_sems[u])

    def wait(c, u):
        pltpu.make_async_copy(hp_s.at[idx_a(c)], a_bufs[u], a_sems[u]).wait()
        pltpu.make_async_copy(emb_s.at[idx_b(c)], b_bufs[u], b_sems[u]).wait()

    def compute(c, u):
        a_v, b_v = a_bufs[u], b_bufs[u]
        lane = lax.iota(jnp.int32, L)
        for g in range(EC // L):
            rows = lane + g * L
            acc0 = jnp.zeros((L,), jnp.float32)
            acc1 = jnp.zeros((L,), jnp.float32)

            def wstep(w8, accs):
                acc0, acc1 = accs
                for uu in range(8):
                    cols = (lane + (w8 * 8 + uu)) & (W - 1)
                    wa = plsc.load_gather(a_v, [rows, cols])
                    wb = plsc.load_gather(b_v, [rows, cols])
                    a_lo, a_hi = plsc.unpack(
                        plsc.bitcast(wa, jnp.bfloat16),
                        format=plsc.PackFormat.INTERLEAVED)
                    b_lo, b_hi = plsc.unpack(
                        plsc.bitcast(wb, jnp.bfloat16),
                        format=plsc.PackFormat.INTERLEAVED)
                    acc0 = acc0 + a_lo * b_lo
                    acc1 = acc1 + a_hi * b_hi
                return acc0, acc1

            acc0, acc1 = lax.fori_loop(0, W // 8, wstep, (acc0, acc1))
            so_bufs[u][pl.ds(g * L, L)] = acc0 + acc1

    def body(c, u, static):
        wait(c, u)
        if static:
            if c >= NBUF:
                pltpu.make_async_copy(so_bufs[u], out_at(c - NBUF),
                                      so_sems[u]).wait()
        else:
            @pl.when(c >= NBUF)
            def _():
                pltpu.make_async_copy(so_bufs[u], out_at(c - NBUF),
                                      so_sems[u]).wait()
        compute(c, u)
        pltpu.async_copy(so_bufs[u], out_at(c), so_sems[u])
        if static:
            if c + NBUF < NCHUNK:
                issue(c + NBUF, u)
        else:
            @pl.when(c + NBUF < NCHUNK)
            def _():
                issue(c + NBUF, u)

    for j in range(NBUF):
        issue(j, j)

    FI = NCHUNK // NBUF - 1

    def ring(i2, _):
        for u in range(NBUF):
            body(i2 * NBUF + u, u, static=False)
        return 0

    lax.fori_loop(0, FI, ring, 0)
    for c in range(FI * NBUF, NCHUNK):
        body(c, c % NBUF, static=True)
    for u in range(NBUF):
        c_last = ((NCHUNK - 1 - u) // NBUF) * NBUF + u
        pltpu.make_async_copy(so_bufs[u], out_at(c_last), so_sems[u]).wait()
    _sc3.__exit__(None, None, None)


def kernel(track_emb, edge_index, sampled_tracks):
    track_emb = track_emb.astype(jnp.float32)
    src = edge_index[0].astype(jnp.int32)
    dst = edge_index[1].astype(jnp.int32)
    st = sampled_tracks.astype(jnp.int32)
    s0 = jnp.pad(st[:, 0], (0, P_PAD - P))
    s1 = jnp.pad(st[:, 1], (0, P_PAD - P))
    return _fused_kernel(track_emb, s0, s1, src, dst)


# speedup vs baseline: 1.2426x; 1.0590x over previous
import functools

import jax
import jax.numpy as jnp
from jax import lax
from jax.experimental import pallas as pl
from jax.experimental.pallas import tpu as pltpu
from jax.experimental.pallas import tpu_sc as plsc

P = 10000
T = 10000
E = 320000
D = 128
W = D // 2
NC, NS, L = 2, 16, 16
NW = NC * NS

PK_CH = 25
PK_SUB = T // NS

P_PAD = 10240
HP_SUB = P_PAD // NS
RSUB = 80

EW = E // NW
EC = 80
NCHUNK = EW // EC
NBUF = 2


@functools.partial(
    pl.kernel,
    mesh=plsc.VectorSubcoreMesh(core_axis_name="c", subcore_axis_name="s"),
    compiler_params=pltpu.CompilerParams(needs_layout_passes=False,
                                         use_tc_tiling_on_sc=False,
                                         internal_scratch_in_bytes=4096),
    out_type=jax.ShapeDtypeStruct((E,), jnp.float32),
    scratch_types=[
        pltpu.VMEM_SHARED((T, W), jnp.int32),
        pltpu.VMEM_SHARED((P_PAD, W), jnp.int32),
        pltpu.VMEM((HP_SUB,), jnp.int32),
        pltpu.VMEM((HP_SUB,), jnp.int32),
        pltpu.VMEM((EW,), jnp.int32),
        pltpu.VMEM((EW,), jnp.int32),
        *([pltpu.VMEM((EC, W), jnp.int32)] * 4),
        *([pltpu.VMEM((EC,), jnp.float32)] * 2),
        *([pltpu.SemaphoreType.DMA] * 6),
    ],
)
def _fused_kernel(emb, s0, s1, src, dst, out,
                  emb_s, hp_s, i0_v, i1_v, src_v, dst_v,
                  a0, a1, b0, b1, so0, so1,
                  sa0, sa1, sb0, sb1, so_s0, so_s1):
    sid = lax.axis_index("s")
    cid = lax.axis_index("c")
    wid = sid * NC + cid

    PK_SZ = [62] * 10 + [5]

    def pack_phase(pk_f):
        r_off = 0
        for sz in PK_SZ:
            r0 = sid * PK_SUB + r_off
            r_off += sz
            pltpu.sync_copy(emb.at[pl.ds(r0, sz)], pk_f.at[pl.ds(0, sz)])

            def prow(r, _):
                for q in range(D // (2 * L)):
                    pair = plsc.pack(
                        pk_f[r, pl.ds(q * 2 * L, L)],
                        pk_f[r, pl.ds(q * 2 * L + L, L)],
                        format=plsc.PackFormat.INTERLEAVED)
                    b0[r, pl.ds(q * L, L)] = plsc.bitcast(pair, jnp.int32)
                return 0

            lax.fori_loop(0, sz, prow, 0)
            pltpu.sync_copy(b0.at[pl.ds(0, sz)], emb_s.at[pl.ds(r0, sz)])

    pl.run_scoped(pack_phase, pltpu.VMEM((62, D), jnp.float32))
    plsc.subcore_barrier()
    _sc2 = jax.named_scope("hplay_phase"); _sc2.__enter__()

    hb = sid * HP_SUB
    pltpu.sync_copy(s0.at[pl.ds(hb, HP_SUB)], i0_v)
    pltpu.sync_copy(s1.at[pl.ds(hb, HP_SUB)], i1_v)
    half = jnp.full((2 * L,), 0.5, jnp.bfloat16)
    for k in range(HP_SUB // RSUB):
        ca = pltpu.async_copy(emb_s.at[i0_v.at[pl.ds(k * RSUB, RSUB)]], a0, sa0)
        cb = pltpu.async_copy(emb_s.at[i1_v.at[pl.ds(k * RSUB, RSUB)]], b0, sb0)
        ca.wait()
        cb.wait()

        def hrow(r, _):
            for q in range(W // L):
                sl = pl.ds(q * L, L)
                m = (plsc.bitcast(a0[r, sl], jnp.bfloat16) +
                     plsc.bitcast(b0[r, sl], jnp.bfloat16)) * half
                a0[r, sl] = plsc.bitcast(m, jnp.int32)
            return 0

        lax.fori_loop(0, RSUB, hrow, 0)
        pltpu.sync_copy(a0, hp_s.at[pl.ds(hb + k * RSUB, RSUB)])
    _sc2.__exit__(None, None, None)
    plsc.subcore_barrier()
    _sc3 = jax.named_scope("score_phase"); _sc3.__enter__()

    eb = wid * EW
    pltpu.sync_copy(src.at[pl.ds(eb, EW)], src_v)
    pltpu.sync_copy(dst.at[pl.ds(eb, EW)], dst_v)

    a_bufs, b_bufs = (a0, a1), (b0, b1)
    a_sems, b_sems = (sa0, sa1), (sb0, sb1)
    so_bufs, so_sems = (so0, so1), (so_s0, so_s1)

    def idx_a(c):
        return src_v.at[pl.ds(pl.multiple_of(c * EC, 8), EC)]

    def idx_b(c):
        return dst_v.at[pl.ds(pl.multiple_of(c * EC, 8), EC)]

    def out_at(c):
        return out.at[pl.ds(eb + pl.multiple_of(c * EC, 8), EC)]

    def issue(c, u):
        pltpu.async_copy(hp_s.at[idx_a(c)], a_bufs[u], a_sems[u])
        pltpu.async_copy(emb_s.at[idx_b(c)], b_bufs[u], b_sems[u])

    def wait(c, u):
        pltpu.make_async_copy(hp_s.at[idx_a(c)], a_bufs[u], a_sems[u]).wait()
        pltpu.make_async_copy(emb_s.at[idx_b(c)], b_bufs[u], b_sems[u]).wait()

    def compute(c, u):
        a_v, b_v = a_bufs[u], b_bufs[u]
        lane = lax.iota(jnp.int32, L)
        for g in range(EC // L):
            rows = lane + g * L
            acc0 = jnp.zeros((L,), jnp.float32)
            acc1 = jnp.zeros((L,), jnp.float32)

            def wstep(w8, accs):
                acc0, acc1 = accs
                for uu in range(8):
                    cols = (lane + (w8 * 8 + uu)) & (W - 1)
                    wa = plsc.load_gather(a_v, [rows, cols])
                    wb = plsc.load_gather(b_v, [rows, cols])
                    a_lo, a_hi = plsc.unpack(
                        plsc.bitcast(wa, jnp.bfloat16),
                        format=plsc.PackFormat.INTERLEAVED)
                    b_lo, b_hi = plsc.unpack(
                        plsc.bitcast(wb, jnp.bfloat16),
                        format=plsc.PackFormat.INTERLEAVED)
                    acc0 = acc0 + a_lo * b_lo
                    acc1 = acc1 + a_hi * b_hi
                return acc0, acc1

            acc0, acc1 = lax.fori_loop(0, W // 8, wstep, (acc0, acc1))
            so_bufs[u][pl.ds(g * L, L)] = acc0 + acc1

    def body(c, u, static):
        wait(c, u)
        if static:
            if c >= NBUF:
                pltpu.make_async_copy(so_bufs[u], out_at(c - NBUF),
                                      so_sems[u]).wait()
        else:
            @pl.when(c >= NBUF)
            def _():
                pltpu.make_async_copy(so_bufs[u], out_at(c - NBUF),
                                      so_sems[u]).wait()
        compute(c, u)
        pltpu.async_copy(so_bufs[u], out_at(c), so_sems[u])
        if static:
            if c + NBUF < NCHUNK:
                issue(c + NBUF, u)
        else:
            @pl.when(c + NBUF < NCHUNK)
            def _():
                issue(c + NBUF, u)

    for j in range(NBUF):
        issue(j, j)

    FI = NCHUNK // NBUF - 1

    def ring(i2, _):
        for u in range(NBUF):
            body(i2 * NBUF + u, u, static=False)
        return 0

    lax.fori_loop(0, FI, ring, 0)
    for c in range(FI * NBUF, NCHUNK):
        body(c, c % NBUF, static=True)
    for u in range(NBUF):
        c_last = ((NCHUNK - 1 - u) // NBUF) * NBUF + u
        pltpu.make_async_copy(so_bufs[u], out_at(c_last), so_sems[u]).wait()
    _sc3.__exit__(None, None, None)


def kernel(track_emb, edge_index, sampled_tracks):
    track_emb = track_emb.astype(jnp.float32)
    src = edge_index[0].astype(jnp.int32)
    dst = edge_index[1].astype(jnp.int32)
    st = sampled_tracks.astype(jnp.int32)
    s0 = jnp.pad(st[:, 0], (0, P_PAD - P))
    s1 = jnp.pad(st[:, 1], (0, P_PAD - P))
    return _fused_kernel(track_emb, s0, s1, src, dst)


# speedup vs baseline: 1.3592x; 1.0938x over previous
import functools

import jax
import jax.numpy as jnp
from jax import lax
from jax.experimental import pallas as pl
from jax.experimental.pallas import tpu as pltpu
from jax.experimental.pallas import tpu_sc as plsc

P = 10000
T = 10000
E = 320000
D = 128
W = D // 2
NC, NS, L = 2, 16, 16
NW = NC * NS

PK_CH = 25
PK_SUB = T // NS

P_PAD = 10240
HP_SUB = P_PAD // NS
RSUB = 80

EW = E // NW
EC = 80
NCHUNK = EW // EC
NBUF = 2


@functools.partial(
    pl.kernel,
    mesh=plsc.VectorSubcoreMesh(core_axis_name="c", subcore_axis_name="s"),
    compiler_params=pltpu.CompilerParams(needs_layout_passes=False,
                                         use_tc_tiling_on_sc=False,
                                         internal_scratch_in_bytes=4096),
    out_type=jax.ShapeDtypeStruct((E,), jnp.float32),
    scratch_types=[
        pltpu.VMEM_SHARED((T, W), jnp.int32),
        pltpu.VMEM_SHARED((P_PAD, W), jnp.int32),
        pltpu.VMEM((HP_SUB,), jnp.int32),
        pltpu.VMEM((HP_SUB,), jnp.int32),
        pltpu.VMEM((EW,), jnp.int32),
        pltpu.VMEM((EW,), jnp.int32),
        *([pltpu.VMEM((EC, W), jnp.int32)] * 4),
        *([pltpu.VMEM((EC,), jnp.float32)] * 2),
        *([pltpu.SemaphoreType.DMA] * 8),
    ],
)
def _fused_kernel(emb, s0, s1, src, dst, out,
                  emb_s, hp_s, i0_v, i1_v, src_v, dst_v,
                  a0, a1, b0, b1, so0, so1,
                  sa0, sa1, sb0, sb1, so_s0, so_s1, spk0, spk1):
    sid = lax.axis_index("s")
    cid = lax.axis_index("c")
    wid = sid * NC + cid

    eb = wid * EW
    hb = sid * HP_SUB
    h_src = pltpu.async_copy(src.at[pl.ds(eb, EW)], src_v, so_s0)
    h_dst = pltpu.async_copy(dst.at[pl.ds(eb, EW)], dst_v, so_s1)
    h_s0 = pltpu.async_copy(s0.at[pl.ds(hb, HP_SUB)], i0_v, sa0)
    h_s1 = pltpu.async_copy(s1.at[pl.ds(hb, HP_SUB)], i1_v, sb0)

    PK_SZ = [31] * 20 + [5]
    PK_OFF = [31 * i for i in range(21)]

    def pack_phase(pf0, pf1):
        pfs, pk_sems = (pf0, pf1), (spk0, spk1)
        hs = {}

        def pk_issue(ci):
            u, sz = ci % 2, PK_SZ[ci]
            hs[ci] = pltpu.async_copy(
                emb.at[pl.ds(sid * PK_SUB + PK_OFF[ci], sz)],
                pfs[u].at[pl.ds(0, sz)], pk_sems[u])

        pk_issue(0)
        pk_issue(1)
        for ci, sz in enumerate(PK_SZ):
            hs[ci].wait()
            pf = pfs[ci % 2]

            def prow(r, _):
                for q in range(D // (2 * L)):
                    pair = plsc.pack(
                        pf[r, pl.ds(q * 2 * L, L)],
                        pf[r, pl.ds(q * 2 * L + L, L)],
                        format=plsc.PackFormat.INTERLEAVED)
                    b0[r, pl.ds(q * L, L)] = plsc.bitcast(pair, jnp.int32)
                return 0

            lax.fori_loop(0, sz, prow, 0)
            pltpu.sync_copy(b0.at[pl.ds(0, sz)],
                            emb_s.at[pl.ds(sid * PK_SUB + PK_OFF[ci], sz)])
            if ci + 2 < len(PK_SZ):
                pk_issue(ci + 2)

    pl.run_scoped(pack_phase,
                  pltpu.VMEM((31, D), jnp.float32),
                  pltpu.VMEM((31, D), jnp.float32))
    plsc.subcore_barrier()
    _sc2 = jax.named_scope("hplay_phase"); _sc2.__enter__()

    h_s0.wait()
    h_s1.wait()
    half = jnp.full((2 * L,), 0.5, jnp.bfloat16)
    HP_N = HP_SUB // RSUB
    hp_a, hp_b = (a0, a1), (b0, b1)
    hp_sa, hp_sb = (sa0, sa1), (sb0, sb1)
    hps = {}

    def hp_issue(k):
        u = k % 2
        hps[k] = (
            pltpu.async_copy(emb_s.at[i0_v.at[pl.ds(k * RSUB, RSUB)]],
                             hp_a[u], hp_sa[u]),
            pltpu.async_copy(emb_s.at[i1_v.at[pl.ds(k * RSUB, RSUB)]],
                             hp_b[u], hp_sb[u]),
        )

    hp_issue(0)
    hp_issue(1)
    for k in range(HP_N):
        ca, cb = hps[k]
        ca.wait()
        cb.wait()
        av, bv = hp_a[k % 2], hp_b[k % 2]

        def hrow(r, _):
            for q in range(W // L):
                sl = pl.ds(q * L, L)
                m = (plsc.bitcast(av[r, sl], jnp.bfloat16) +
                     plsc.bitcast(bv[r, sl], jnp.bfloat16)) * half
                av[r, sl] = plsc.bitcast(m, jnp.int32)
            return 0

        lax.fori_loop(0, RSUB, hrow, 0)
        pltpu.sync_copy(av, hp_s.at[pl.ds(hb + k * RSUB, RSUB)])
        if k + 2 < HP_N:
            hp_issue(k + 2)
    _sc2.__exit__(None, None, None)
    plsc.subcore_barrier()
    _sc3 = jax.named_scope("score_phase"); _sc3.__enter__()

    h_src.wait()
    h_dst.wait()

    a_bufs, b_bufs = (a0, a1), (b0, b1)
    a_sems, b_sems = (sa0, sa1), (sb0, sb1)
    so_bufs, so_sems = (so0, so1), (so_s0, so_s1)

    def idx_a(c):
        return src_v.at[pl.ds(pl.multiple_of(c * EC, 8), EC)]

    def idx_b(c):
        return dst_v.at[pl.ds(pl.multiple_of(c * EC, 8), EC)]

    def out_at(c):
        return out.at[pl.ds(eb + pl.multiple_of(c * EC, 8), EC)]

    def issue(c, u):
        pltpu.async_copy(hp_s.at[idx_a(c)], a_bufs[u], a_sems[u])
        pltpu.async_copy(emb_s.at[idx_b(c)], b_bufs[u], b_sems[u])

    def wait(c, u):
        pltpu.make_async_copy(hp_s.at[idx_a(c)], a_bufs[u], a_sems[u]).wait()
        pltpu.make_async_copy(emb_s.at[idx_b(c)], b_bufs[u], b_sems[u]).wait()

    def compute(c, u):
        a_v, b_v = a_bufs[u], b_bufs[u]
        lane = lax.iota(jnp.int32, L)
        for g in range(EC // L):
            rows = lane + g * L
            acc0 = jnp.zeros((L,), jnp.float32)
            acc1 = jnp.zeros((L,), jnp.float32)

            def wstep(w8, accs):
                acc0, acc1 = accs
                for uu in range(8):
                    cols = (lane + (w8 * 8 + uu)) & (W - 1)
                    wa = plsc.load_gather(a_v, [rows, cols])
                    wb = plsc.load_gather(b_v, [rows, cols])
                    a_lo, a_hi = plsc.unpack(
                        plsc.bitcast(wa, jnp.bfloat16),
                        format=plsc.PackFormat.INTERLEAVED)
                    b_lo, b_hi = plsc.unpack(
                        plsc.bitcast(wb, jnp.bfloat16),
                        format=plsc.PackFormat.INTERLEAVED)
                    acc0 = acc0 + a_lo * b_lo
                    acc1 = acc1 + a_hi * b_hi
                return acc0, acc1

            acc0, acc1 = lax.fori_loop(0, W // 8, wstep, (acc0, acc1))
            so_bufs[u][pl.ds(g * L, L)] = acc0 + acc1

    def body(c, u, static):
        wait(c, u)
        if static:
            if c >= NBUF:
                pltpu.make_async_copy(so_bufs[u], out_at(c - NBUF),
                                      so_sems[u]).wait()
        else:
            @pl.when(c >= NBUF)
            def _():
                pltpu.make_async_copy(so_bufs[u], out_at(c - NBUF),
                                      so_sems[u]).wait()
        compute(c, u)
        pltpu.async_copy(so_bufs[u], out_at(c), so_sems[u])
        if static:
            if c + NBUF < NCHUNK:
                issue(c + NBUF, u)
        else:
            @pl.when(c + NBUF < NCHUNK)
            def _():
                issue(c + NBUF, u)

    for j in range(NBUF):
        issue(j, j)

    FI = NCHUNK // NBUF - 1

    def ring(i2, _):
        for u in range(NBUF):
            body(i2 * NBUF + u, u, static=False)
        return 0

    lax.fori_loop(0, FI, ring, 0)
    for c in range(FI * NBUF, NCHUNK):
        body(c, c % NBUF, static=True)
    for u in range(NBUF):
        c_last = ((NCHUNK - 1 - u) // NBUF) * NBUF + u
        pltpu.make_async_copy(so_bufs[u], out_at(c_last), so_sems[u]).wait()
    _sc3.__exit__(None, None, None)


def kernel(track_emb, edge_index, sampled_tracks):
    track_emb = track_emb.astype(jnp.float32)
    src = edge_index[0].astype(jnp.int32)
    dst = edge_index[1].astype(jnp.int32)
    st = sampled_tracks.astype(jnp.int32)
    s0 = jnp.pad(st[:, 0], (0, P_PAD - P))
    s1 = jnp.pad(st[:, 1], (0, P_PAD - P))
    return _fused_kernel(track_emb, s0, s1, src, dst)


# speedup vs baseline: 1.5384x; 1.1319x over previous
import functools

import jax
import jax.numpy as jnp
from jax import lax
from jax.experimental import pallas as pl
from jax.experimental.pallas import tpu as pltpu
from jax.experimental.pallas import tpu_sc as plsc

P = 10000
T = 10000
E = 320000
D = 128
W = D // 2
NC, NS, L = 2, 16, 16
NW = NC * NS

PK_CH = 25
PK_SUB = T // NS

P_PAD = 10240
HP_SUB = P_PAD // NS
RSUB = 80

EW = E // NW
EC = 80
NCHUNK = EW // EC
NBUF = 2


@functools.partial(
    pl.kernel,
    mesh=plsc.VectorSubcoreMesh(core_axis_name="c", subcore_axis_name="s"),
    compiler_params=pltpu.CompilerParams(needs_layout_passes=False,
                                         use_tc_tiling_on_sc=False,
                                         internal_scratch_in_bytes=4096),
    out_type=jax.ShapeDtypeStruct((E,), jnp.float32),
    scratch_types=[
        pltpu.VMEM_SHARED((T, W), jnp.int32),
        pltpu.VMEM_SHARED((P_PAD, W), jnp.int32),
        pltpu.VMEM((HP_SUB,), jnp.int32),
        pltpu.VMEM((HP_SUB,), jnp.int32),
        pltpu.VMEM((EW,), jnp.int32),
        pltpu.VMEM((EW,), jnp.int32),
        *([pltpu.VMEM((EC, W), jnp.int32)] * 4),
        *([pltpu.VMEM((EC,), jnp.float32)] * 2),
        *([pltpu.SemaphoreType.DMA] * 8),
    ],
)
def _fused_kernel(emb, s0, s1, src, dst, out,
                  emb_s, hp_s, i0_v, i1_v, src_v, dst_v,
                  a0, a1, b0, b1, so0, so1,
                  sa0, sa1, sb0, sb1, so_s0, so_s1, spk0, spk1):
    sid = lax.axis_index("s")
    cid = lax.axis_index("c")
    wid = sid * NC + cid

    eb = wid * EW
    hb = sid * HP_SUB
    h_src = pltpu.async_copy(src.at[pl.ds(eb, EW)], src_v, so_s0)
    h_dst = pltpu.async_copy(dst.at[pl.ds(eb, EW)], dst_v, so_s1)
    h_s0 = pltpu.async_copy(s0.at[pl.ds(hb, HP_SUB)], i0_v, sa0)
    h_s1 = pltpu.async_copy(s1.at[pl.ds(hb, HP_SUB)], i1_v, sb0)

    PK_SZ = [31] * 20 + [5]
    PK_OFF = [31 * i for i in range(21)]

    def pack_phase(pf0, pf1):
        pfs, pk_sems = (pf0, pf1), (spk0, spk1)
        hs = {}

        def pk_issue(ci):
            u, sz = ci % 2, PK_SZ[ci]
            hs[ci] = pltpu.async_copy(
                emb.at[pl.ds(sid * PK_SUB + PK_OFF[ci], sz)],
                pfs[u].at[pl.ds(0, sz)], pk_sems[u])

        pk_issue(0)
        pk_issue(1)
        for ci, sz in enumerate(PK_SZ):
            hs[ci].wait()
            pf = pfs[ci % 2]

            def prow(r, _):
                for q in range(D // (2 * L)):
                    pair = plsc.pack(
                        pf[r, pl.ds(q * 2 * L, L)],
                        pf[r, pl.ds(q * 2 * L + L, L)],
                        format=plsc.PackFormat.INTERLEAVED)
                    b0[r, pl.ds(q * L, L)] = plsc.bitcast(pair, jnp.int32)
                return 0

            lax.fori_loop(0, sz, prow, 0)
            pltpu.sync_copy(b0.at[pl.ds(0, sz)],
                            emb_s.at[pl.ds(sid * PK_SUB + PK_OFF[ci], sz)])
            if ci + 2 < len(PK_SZ):
                pk_issue(ci + 2)

    pl.run_scoped(pack_phase,
                  pltpu.VMEM((31, D), jnp.float32),
                  pltpu.VMEM((31, D), jnp.float32))
    plsc.subcore_barrier()
    _sc2 = jax.named_scope("hplay_phase"); _sc2.__enter__()

    h_s0.wait()
    h_s1.wait()
    half = jnp.full((2 * L,), 0.5, jnp.bfloat16)
    HP_N = HP_SUB // RSUB
    hp_a, hp_b = (a0, a1), (b0, b1)
    hp_sa, hp_sb = (sa0, sa1), (sb0, sb1)
    hps = {}

    def hp_issue(k):
        u = k % 2
        hps[k] = (
            pltpu.async_copy(emb_s.at[i0_v.at[pl.ds(k * RSUB, RSUB)]],
                             hp_a[u], hp_sa[u]),
            pltpu.async_copy(emb_s.at[i1_v.at[pl.ds(k * RSUB, RSUB)]],
                             hp_b[u], hp_sb[u]),
        )

    hp_issue(0)
    hp_issue(1)
    for k in range(HP_N):
        ca, cb = hps[k]
        ca.wait()
        cb.wait()
        av, bv = hp_a[k % 2], hp_b[k % 2]

        def hrow(r, _):
            for q in range(W // L):
                sl = pl.ds(q * L, L)
                m = (plsc.bitcast(av[r, sl], jnp.bfloat16) +
                     plsc.bitcast(bv[r, sl], jnp.bfloat16)) * half
                av[r, sl] = plsc.bitcast(m, jnp.int32)
            return 0

        lax.fori_loop(0, RSUB, hrow, 0)
        pltpu.sync_copy(av, hp_s.at[pl.ds(hb + k * RSUB, RSUB)])
        if k + 2 < HP_N:
            hp_issue(k + 2)
    _sc2.__exit__(None, None, None)
    plsc.subcore_barrier()
    _sc3 = jax.named_scope("score_phase"); _sc3.__enter__()

    h_src.wait()
    h_dst.wait()

    a_bufs, b_bufs = (a0, a1), (b0, b1)
    a_sems, b_sems = (sa0, sa1), (sb0, sb1)
    so_bufs, so_sems = (so0, so1), (so_s0, so_s1)

    def idx_a(c):
        return src_v.at[pl.ds(pl.multiple_of(c * EC, 8), EC)]

    def idx_b(c):
        return dst_v.at[pl.ds(pl.multiple_of(c * EC, 8), EC)]

    def out_at(c):
        return out.at[pl.ds(eb + pl.multiple_of(c * EC, 8), EC)]

    def issue(c, u):
        pltpu.async_copy(hp_s.at[idx_a(c)], a_bufs[u], a_sems[u])
        pltpu.async_copy(emb_s.at[idx_b(c)], b_bufs[u], b_sems[u])

    def wait(c, u):
        pltpu.make_async_copy(hp_s.at[idx_a(c)], a_bufs[u], a_sems[u]).wait()
        pltpu.make_async_copy(emb_s.at[idx_b(c)], b_bufs[u], b_sems[u]).wait()

    def compute(c, u):
        a_v, b_v = a_bufs[u], b_bufs[u]
        lane = lax.iota(jnp.int32, L)
        for g in range(EC // L):
            rows = lane + g * L
            acc0 = jnp.zeros((L,), jnp.float32)
            acc1 = jnp.zeros((L,), jnp.float32)

            def wstep(w8, accs):
                acc0, acc1 = accs
                for uu in range(8):
                    cols = (lane + (w8 * 8 + uu)) & (W - 1)
                    wa = plsc.load_gather(a_v, [rows, cols])
                    wb = plsc.load_gather(b_v, [rows, cols])
                    wp = (plsc.bitcast(wa, jnp.bfloat16) *
                          plsc.bitcast(wb, jnp.bfloat16))
                    p_lo, p_hi = plsc.unpack(
                        wp, format=plsc.PackFormat.INTERLEAVED)
                    acc0 = acc0 + p_lo
                    acc1 = acc1 + p_hi
                return acc0, acc1

            acc0, acc1 = lax.fori_loop(0, W // 8, wstep, (acc0, acc1))
            so_bufs[u][pl.ds(g * L, L)] = acc0 + acc1

    def body(c, u, static):
        wait(c, u)
        if static:
            if c >= NBUF:
                pltpu.make_async_copy(so_bufs[u], out_at(c - NBUF),
                                      so_sems[u]).wait()
        else:
            @pl.when(c >= NBUF)
            def _():
                pltpu.make_async_copy(so_bufs[u], out_at(c - NBUF),
                                      so_sems[u]).wait()
        compute(c, u)
        pltpu.async_copy(so_bufs[u], out_at(c), so_sems[u])
        if static:
            if c + NBUF < NCHUNK:
                issue(c + NBUF, u)
        else:
            @pl.when(c + NBUF < NCHUNK)
            def _():
                issue(c + NBUF, u)

    for j in range(NBUF):
        issue(j, j)

    FI = NCHUNK // NBUF - 1

    def ring(i2, _):
        for u in range(NBUF):
            body(i2 * NBUF + u, u, static=False)
        return 0

    lax.fori_loop(0, FI, ring, 0)
    for c in range(FI * NBUF, NCHUNK):
        body(c, c % NBUF, static=True)
    for u in range(NBUF):
        c_last = ((NCHUNK - 1 - u) // NBUF) * NBUF + u
        pltpu.make_async_copy(so_bufs[u], out_at(c_last), so_sems[u]).wait()
    _sc3.__exit__(None, None, None)


def kernel(track_emb, edge_index, sampled_tracks):
    track_emb = track_emb.astype(jnp.float32)
    src = edge_index[0].astype(jnp.int32)
    dst = edge_index[1].astype(jnp.int32)
    st = sampled_tracks.astype(jnp.int32)
    s0 = jnp.pad(st[:, 0], (0, P_PAD - P))
    s1 = jnp.pad(st[:, 1], (0, P_PAD - P))
    return _fused_kernel(track_emb, s0, s1, src, dst)
